# baseline (speedup 1.0000x reference)
import functools

import jax
import jax.numpy as jnp
from jax import lax
from jax.experimental import pallas as pl
from jax.experimental.pallas import tpu as pltpu
from jax.experimental.pallas import tpu_sc as plsc

VOCAB = 1000000
BATCH = 16384
FIELDS = 100

NC = 2
NS = 16
L = 16
NW = NC * NS

ROWS_PER_W = BATCH // NW
CH_ROWS = 64
N_CH = ROWS_PER_W // CH_ROWS
CHF = CH_ROWS * FIELDS

SHARD = 62496
HOP = 12800
TAIL = SHARD - 4 * HOP
TAIL_LAST = VOCAB - 15 * SHARD - 4 * HOP

COL_STARTS = (0, 16, 32, 48, 64, 80, 84)


def _lookup_body(x_hbm, table_hbm, out_hbm, shared_tab,
                 idx_a, idx_b, val_a, val_b, bnc_a, bnc_b, sem_t, sem_t2,
                 sem_l, sem_g, sem_o):
    wid = lax.axis_index("s") * NC + lax.axis_index("c")
    sid = lax.axis_index("s")
    base = wid * ROWS_PER_W
    idx_bufs = (idx_a, idx_b)
    val_bufs = (val_a, val_b)
    shard_off = sid * SHARD

    def hop_src(h):
        return table_hbm.at[pl.ds(shard_off + h * HOP, HOP)]

    def hop_dst(h):
        return shared_tab.at[pl.ds(shard_off + h * HOP, HOP)]

    def x_slice(c):
        return x_hbm.at[pl.ds(base + c * CH_ROWS, CH_ROWS)]

    def out_slice(c):
        return out_hbm.at[pl.ds(base + c * CH_ROWS, CH_ROWS)]

    def mod_chunk(idx_v):
        def mod_row(j, carry):
            for c0 in COL_STARTS:
                s = pl.ds(c0, L)
                v = idx_v[j, s]
                idx_v[j, s] = jnp.where(v >= VOCAB, v - VOCAB, v)
            return carry
        lax.fori_loop(0, CH_ROWS, mod_row, 0)

    def fire_rows(idx_v, val_v):
        def fire(j, carry):
            pltpu.async_copy(shared_tab.at[idx_v.at[j]],
                             val_v.at[j], sem_g)
            return carry
        lax.fori_loop(0, CH_ROWS, fire, 0)

    def mod_fire_rows(idx_v, val_v):
        def mf(i, carry):
            for u in range(2):
                j = i * 2 + u
                for c0 in COL_STARTS:
                    s = pl.ds(c0, L)
                    v = idx_v[j, s]
                    idx_v[j, s] = jnp.where(v >= VOCAB, v - VOCAB, v)
                pltpu.async_copy(shared_tab.at[idx_v.at[j]],
                                 val_v.at[j], sem_g)
            return carry
        lax.fori_loop(0, CH_ROWS // 2, mf, 0)

    def drain_cvt(idx_v, val_v):
        def wait_cvt(i, carry):
            for u in range(2):
                j = i * 2 + u
                pltpu.make_async_copy(
                    shared_tab.at[idx_v.at[j]],
                    val_v.at[j], sem_g).wait()
                for c0 in COL_STARTS:
                    idx_v[j, pl.ds(c0, L)] = (
                        val_v[j, pl.ds(c0, L)].astype(jnp.int32))
            return carry
        lax.fori_loop(0, CH_ROWS // 2, wait_cvt, 0)

    pltpu.async_copy(x_slice(0), idx_a, sem_l)
    pltpu.async_copy(hop_src(0), bnc_a, sem_t)
    pltpu.async_copy(hop_src(1), bnc_b, sem_t)

    pltpu.make_async_copy(x_slice(0), idx_a, sem_l).wait()
    mod_chunk(idx_a)

    pltpu.make_async_copy(hop_src(0), bnc_a, sem_t).wait()
    pltpu.async_copy(bnc_a, hop_dst(0), sem_t2)
    pltpu.make_async_copy(hop_src(1), bnc_b, sem_t).wait()
    pltpu.async_copy(bnc_b, hop_dst(1), sem_t2)
    pltpu.make_async_copy(bnc_a, hop_dst(0), sem_t2).wait()
    pltpu.async_copy(hop_src(2), bnc_a, sem_t)
    pltpu.make_async_copy(bnc_b, hop_dst(1), sem_t2).wait()
    pltpu.async_copy(hop_src(3), bnc_b, sem_t)
    pltpu.make_async_copy(hop_src(2), bnc_a, sem_t).wait()
    pltpu.async_copy(bnc_a, hop_dst(2), sem_t2)
    pltpu.make_async_copy(hop_src(3), bnc_b, sem_t).wait()
    pltpu.async_copy(bnc_b, hop_dst(3), sem_t2)
    pltpu.make_async_copy(bnc_a, hop_dst(2), sem_t2).wait()

    tail_off = shard_off + 4 * HOP

    @pl.when(sid < NS - 1)
    def _():
        pltpu.async_copy(table_hbm.at[pl.ds(tail_off, TAIL)],
                         bnc_a.at[pl.ds(0, TAIL)], sem_t)
        pltpu.make_async_copy(table_hbm.at[pl.ds(tail_off, TAIL)],
                              bnc_a.at[pl.ds(0, TAIL)], sem_t).wait()
        pltpu.async_copy(bnc_a.at[pl.ds(0, TAIL)],
                         shared_tab.at[pl.ds(tail_off, TAIL)], sem_t2)
        pltpu.make_async_copy(bnc_a.at[pl.ds(0, TAIL)],
                              shared_tab.at[pl.ds(tail_off, TAIL)],
                              sem_t2).wait()

    @pl.when(sid == NS - 1)
    def _():
        pltpu.async_copy(table_hbm.at[pl.ds(tail_off, TAIL_LAST)],
                         bnc_a.at[pl.ds(0, TAIL_LAST)], sem_t)
        pltpu.make_async_copy(table_hbm.at[pl.ds(tail_off, TAIL_LAST)],
                              bnc_a.at[pl.ds(0, TAIL_LAST)], sem_t).wait()
        pltpu.async_copy(bnc_a.at[pl.ds(0, TAIL_LAST)],
                         shared_tab.at[pl.ds(tail_off, TAIL_LAST)], sem_t2)
        pltpu.make_async_copy(bnc_a.at[pl.ds(0, TAIL_LAST)],
                              shared_tab.at[pl.ds(tail_off, TAIL_LAST)],
                              sem_t2).wait()

    pltpu.make_async_copy(bnc_b, hop_dst(3), sem_t2).wait()
    plsc.subcore_barrier()

    fire_rows(idx_a, val_a)
    pltpu.async_copy(x_slice(1), idx_b, sem_l)

    for c in range(N_CH):
        idx_c, val_c = idx_bufs[c % 2], val_bufs[c % 2]
        if c + 1 < N_CH:
            idx_n, val_n = idx_bufs[(c + 1) % 2], val_bufs[(c + 1) % 2]
            pltpu.make_async_copy(x_slice(c + 1), idx_n, sem_l).wait()
            mod_fire_rows(idx_n, val_n)
        drain_cvt(idx_c, val_c)
        pltpu.async_copy(idx_c, out_slice(c), sem_o)
        if c + 2 < N_CH:
            pltpu.make_async_copy(idx_c, out_slice(c), sem_o).wait()
            pltpu.async_copy(x_slice(c + 2), idx_c, sem_l)

    pltpu.make_async_copy(idx_bufs[(N_CH - 2) % 2],
                          out_slice(N_CH - 2), sem_o).wait()
    pltpu.make_async_copy(idx_bufs[(N_CH - 1) % 2],
                          out_slice(N_CH - 1), sem_o).wait()


@jax.jit
def _lookup(x, table):
    mesh = plsc.VectorSubcoreMesh(core_axis_name="c", subcore_axis_name="s")
    f = functools.partial(
        pl.kernel,
        mesh=mesh,
        out_type=jax.ShapeDtypeStruct((BATCH, FIELDS), jnp.int32),
        scratch_types=[
            pltpu.VMEM_SHARED((VOCAB,), jnp.float32),
            pltpu.VMEM((CH_ROWS, FIELDS), jnp.int32),
            pltpu.VMEM((CH_ROWS, FIELDS), jnp.int32),
            pltpu.VMEM((CH_ROWS, FIELDS), jnp.float32),
            pltpu.VMEM((CH_ROWS, FIELDS), jnp.float32),
            pltpu.VMEM((HOP,), jnp.float32),
            pltpu.VMEM((HOP,), jnp.float32),
            pltpu.SemaphoreType.DMA,
            pltpu.SemaphoreType.DMA,
            pltpu.SemaphoreType.DMA,
            pltpu.SemaphoreType.DMA,
            pltpu.SemaphoreType.DMA,
        ],
        compiler_params=pltpu.CompilerParams(use_tc_tiling_on_sc=True),
    )(_lookup_body)
    return f(x, table)


def kernel(x, table):
    return _lookup(x, table)


# speedup vs baseline: 1.0066x; 1.0066x over previous
import functools

import jax
import jax.numpy as jnp
from jax import lax
from jax.experimental import pallas as pl
from jax.experimental.pallas import tpu as pltpu
from jax.experimental.pallas import tpu_sc as plsc

VOCAB = 1000000
BATCH = 16384
FIELDS = 100

NC = 2
NS = 16
L = 16
NW = NC * NS

ROWS_PER_W = BATCH // NW
CH_ROWS = 64
N_CH = ROWS_PER_W // CH_ROWS
CHF = CH_ROWS * FIELDS

SHARD = 62496
HOP = 12800
TAIL = SHARD - 4 * HOP
TAIL_LAST = VOCAB - 15 * SHARD - 4 * HOP

COL_STARTS = (0, 16, 32, 48, 64, 80, 84)


def _lookup_body(x_hbm, table_hbm, out_hbm, shared_tab,
                 idx_a, idx_b, val_a, val_b, bnc_a, bnc_b, sem_t, sem_t2,
                 sem_l, sem_g, sem_o):
    wid = lax.axis_index("s") * NC + lax.axis_index("c")
    sid = lax.axis_index("s")
    base = wid * ROWS_PER_W
    idx_bufs = (idx_a, idx_b)
    val_bufs = (val_a, val_b)
    shard_off = sid * SHARD

    def hop_src(h):
        return table_hbm.at[pl.ds(shard_off + h * HOP, HOP)]

    def hop_dst(h):
        return shared_tab.at[pl.ds(shard_off + h * HOP, HOP)]

    def x_slice(c):
        return x_hbm.at[pl.ds(base + c * CH_ROWS, CH_ROWS)]

    def out_slice(c):
        return out_hbm.at[pl.ds(base + c * CH_ROWS, CH_ROWS)]

    def mod_chunk(idx_v):
        def mod_row(j, carry):
            for c0 in COL_STARTS:
                s = pl.ds(c0, L)
                v = lax.bitcast_convert_type(idx_v[j, s], jnp.uint32)
                m = jnp.minimum(v, v - jnp.uint32(VOCAB))
                idx_v[j, s] = lax.bitcast_convert_type(m, jnp.int32)
            return carry
        lax.fori_loop(0, CH_ROWS, mod_row, 0)

    def fire_rows(idx_v, val_v):
        def fire(j, carry):
            pltpu.async_copy(shared_tab.at[idx_v.at[j]],
                             val_v.at[j], sem_g)
            return carry
        lax.fori_loop(0, CH_ROWS, fire, 0)

    def mod_fire_rows(idx_v, val_v):
        def mf(j, carry):
            for c0 in COL_STARTS:
                s = pl.ds(c0, L)
                v = lax.bitcast_convert_type(idx_v[j, s], jnp.uint32)
                m = jnp.minimum(v, v - jnp.uint32(VOCAB))
                idx_v[j, s] = lax.bitcast_convert_type(m, jnp.int32)
            pltpu.async_copy(shared_tab.at[idx_v.at[j]],
                             val_v.at[j], sem_g)
            return carry
        lax.fori_loop(0, CH_ROWS, mf, 0)

    def drain_cvt(idx_v, val_v):
        def wait_cvt(j, carry):
            pltpu.make_async_copy(
                shared_tab.at[idx_v.at[j]],
                val_v.at[j], sem_g).wait()
            for c0 in COL_STARTS:
                idx_v[j, pl.ds(c0, L)] = (
                    val_v[j, pl.ds(c0, L)].astype(jnp.int32))
            return carry
        lax.fori_loop(0, CH_ROWS, wait_cvt, 0)

    pltpu.async_copy(x_slice(0), idx_a, sem_l)
    pltpu.async_copy(hop_src(0), bnc_a, sem_t)
    pltpu.async_copy(hop_src(1), bnc_b, sem_t)

    pltpu.make_async_copy(x_slice(0), idx_a, sem_l).wait()
    mod_chunk(idx_a)

    pltpu.make_async_copy(hop_src(0), bnc_a, sem_t).wait()
    pltpu.async_copy(bnc_a, hop_dst(0), sem_t2)
    pltpu.make_async_copy(hop_src(1), bnc_b, sem_t).wait()
    pltpu.async_copy(bnc_b, hop_dst(1), sem_t2)
    pltpu.make_async_copy(bnc_a, hop_dst(0), sem_t2).wait()
    pltpu.async_copy(hop_src(2), bnc_a, sem_t)
    pltpu.make_async_copy(bnc_b, hop_dst(1), sem_t2).wait()
    pltpu.async_copy(hop_src(3), bnc_b, sem_t)
    pltpu.make_async_copy(hop_src(2), bnc_a, sem_t).wait()
    pltpu.async_copy(bnc_a, hop_dst(2), sem_t2)
    pltpu.make_async_copy(hop_src(3), bnc_b, sem_t).wait()
    pltpu.async_copy(bnc_b, hop_dst(3), sem_t2)
    pltpu.make_async_copy(bnc_a, hop_dst(2), sem_t2).wait()

    tail_off = shard_off + 4 * HOP

    @pl.when(sid < NS - 1)
    def _():
        pltpu.async_copy(table_hbm.at[pl.ds(tail_off, TAIL)],
                         bnc_a.at[pl.ds(0, TAIL)], sem_t)
        pltpu.make_async_copy(table_hbm.at[pl.ds(tail_off, TAIL)],
                              bnc_a.at[pl.ds(0, TAIL)], sem_t).wait()
        pltpu.async_copy(bnc_a.at[pl.ds(0, TAIL)],
                         shared_tab.at[pl.ds(tail_off, TAIL)], sem_t2)
        pltpu.make_async_copy(bnc_a.at[pl.ds(0, TAIL)],
                              shared_tab.at[pl.ds(tail_off, TAIL)],
                              sem_t2).wait()

    @pl.when(sid == NS - 1)
    def _():
        pltpu.async_copy(table_hbm.at[pl.ds(tail_off, TAIL_LAST)],
                         bnc_a.at[pl.ds(0, TAIL_LAST)], sem_t)
        pltpu.make_async_copy(table_hbm.at[pl.ds(tail_off, TAIL_LAST)],
                              bnc_a.at[pl.ds(0, TAIL_LAST)], sem_t).wait()
        pltpu.async_copy(bnc_a.at[pl.ds(0, TAIL_LAST)],
                         shared_tab.at[pl.ds(tail_off, TAIL_LAST)], sem_t2)
        pltpu.make_async_copy(bnc_a.at[pl.ds(0, TAIL_LAST)],
                              shared_tab.at[pl.ds(tail_off, TAIL_LAST)],
                              sem_t2).wait()

    pltpu.make_async_copy(bnc_b, hop_dst(3), sem_t2).wait()
    plsc.subcore_barrier()

    fire_rows(idx_a, val_a)
    pltpu.async_copy(x_slice(1), idx_b, sem_l)

    for c in range(N_CH):
        idx_c, val_c = idx_bufs[c % 2], val_bufs[c % 2]
        if c + 1 < N_CH:
            idx_n, val_n = idx_bufs[(c + 1) % 2], val_bufs[(c + 1) % 2]
            pltpu.make_async_copy(x_slice(c + 1), idx_n, sem_l).wait()
            mod_fire_rows(idx_n, val_n)
        drain_cvt(idx_c, val_c)
        pltpu.async_copy(idx_c, out_slice(c), sem_o)
        if c + 2 < N_CH:
            pltpu.make_async_copy(idx_c, out_slice(c), sem_o).wait()
            pltpu.async_copy(x_slice(c + 2), idx_c, sem_l)

    pltpu.make_async_copy(idx_bufs[(N_CH - 2) % 2],
                          out_slice(N_CH - 2), sem_o).wait()
    pltpu.make_async_copy(idx_bufs[(N_CH - 1) % 2],
                          out_slice(N_CH - 1), sem_o).wait()


@jax.jit
def _lookup(x, table):
    mesh = plsc.VectorSubcoreMesh(core_axis_name="c", subcore_axis_name="s")
    f = functools.partial(
        pl.kernel,
        mesh=mesh,
        out_type=jax.ShapeDtypeStruct((BATCH, FIELDS), jnp.int32),
        scratch_types=[
            pltpu.VMEM_SHARED((VOCAB,), jnp.float32),
            pltpu.VMEM((CH_ROWS, FIELDS), jnp.int32),
            pltpu.VMEM((CH_ROWS, FIELDS), jnp.int32),
            pltpu.VMEM((CH_ROWS, FIELDS), jnp.float32),
            pltpu.VMEM((CH_ROWS, FIELDS), jnp.float32),
            pltpu.VMEM((HOP,), jnp.float32),
            pltpu.VMEM((HOP,), jnp.float32),
            pltpu.SemaphoreType.DMA,
            pltpu.SemaphoreType.DMA,
            pltpu.SemaphoreType.DMA,
            pltpu.SemaphoreType.DMA,
            pltpu.SemaphoreType.DMA,
        ],
        compiler_params=pltpu.CompilerParams(use_tc_tiling_on_sc=True),
    )(_lookup_body)
    return f(x, table)


def kernel(x, table):
    return _lookup(x, table)
